# Initial kernel scaffold; baseline (speedup 1.0000x reference)
import jax
import jax.numpy as jnp
from jax.experimental import pallas as pl


def kernel(x, position_ids, cos_cached, sin_cached):
    raise NotImplementedError("write your pallas kernel here")


# speedup vs baseline: 3.0475x; 3.0475x over previous
import functools

import jax
import jax.numpy as jnp
from jax import lax
from jax.experimental import pallas as pl
from jax.experimental.pallas import tpu as pltpu
from jax.experimental.pallas import tpu_sc as plsc

DIM = 64
CHUNK = 128


@functools.partial(jax.jit, static_argnums=())
def _rope_gather(position_ids, cos_cached, sin_cached):
    n_total = position_ids.size
    info = plsc.get_sparse_core_info()
    nw = info.num_cores * info.num_subcores
    n_per_w = n_total // nw
    n_ch = n_per_w // CHUNK

    idx = position_ids.reshape(nw, n_ch, CHUNK)
    mesh = plsc.VectorSubcoreMesh(core_axis_name="c", subcore_axis_name="s")

    @functools.partial(
        pl.kernel,
        mesh=mesh,
        compiler_params=pltpu.CompilerParams(use_tc_tiling_on_sc=False),
        out_type=(
            jax.ShapeDtypeStruct((n_total, DIM), jnp.float32),
            jax.ShapeDtypeStruct((n_total, DIM), jnp.float32),
        ),
        scratch_types=[
            pltpu.VMEM((n_ch, CHUNK), jnp.int32),
            pltpu.VMEM((CHUNK, DIM), jnp.float32),
            pltpu.VMEM((CHUNK, DIM), jnp.float32),
            pltpu.SemaphoreType.DMA,
            pltpu.SemaphoreType.DMA,
        ],
    )
    def k(cos_hbm, sin_hbm, idx_hbm, cos_out, sin_out, idx_v, cbuf, sbuf, csem, ssem):
        wid = lax.axis_index("s") * info.num_cores + lax.axis_index("c")
        base = wid * n_per_w
        pltpu.sync_copy(idx_hbm.at[wid], idx_v)
        for j in range(n_ch):
            cop = pltpu.async_copy(cos_hbm.at[idx_v.at[j]], cbuf, csem)
            sop = pltpu.async_copy(sin_hbm.at[idx_v.at[j]], sbuf, ssem)
            cop.wait()
            pltpu.sync_copy(cbuf, cos_out.at[pl.ds(base + j * CHUNK, CHUNK)])
            sop.wait()
            pltpu.sync_copy(sbuf, sin_out.at[pl.ds(base + j * CHUNK, CHUNK)])

    return k(cos_cached, sin_cached, idx)


def kernel(x, position_ids, cos_cached, sin_cached):
    b, s = position_ids.shape
    cos_flat, sin_flat = _rope_gather(position_ids, cos_cached, sin_cached)
    cos = cos_flat.reshape(b, s, DIM).astype(x.dtype)
    sin = sin_flat.reshape(b, s, DIM).astype(x.dtype)
    return cos, sin


# speedup vs baseline: 3.1860x; 1.0454x over previous
import functools

import jax
import jax.numpy as jnp
from jax import lax
from jax.experimental import pallas as pl
from jax.experimental.pallas import tpu as pltpu
from jax.experimental.pallas import tpu_sc as plsc

DIM = 64
CHUNK = 128


@functools.partial(jax.jit, static_argnums=())
def _rope_gather(position_ids, cos_cached, sin_cached):
    n_total = position_ids.size
    info = plsc.get_sparse_core_info()
    nw = info.num_cores * info.num_subcores
    n_per_w = n_total // nw
    n_ch = n_per_w // CHUNK

    idx = position_ids.reshape(nw, n_ch, CHUNK)
    mesh = plsc.VectorSubcoreMesh(core_axis_name="c", subcore_axis_name="s")

    nbuf = 4
    look = 2

    @functools.partial(
        pl.kernel,
        mesh=mesh,
        compiler_params=pltpu.CompilerParams(use_tc_tiling_on_sc=False),
        out_type=(
            jax.ShapeDtypeStruct((n_total, DIM), jnp.float32),
            jax.ShapeDtypeStruct((n_total, DIM), jnp.float32),
        ),
        scratch_types=[
            pltpu.VMEM((n_ch, CHUNK), jnp.int32),
            pltpu.VMEM((nbuf, CHUNK, DIM), jnp.float32),
            pltpu.VMEM((nbuf, CHUNK, DIM), jnp.float32),
            pltpu.SemaphoreType.DMA((nbuf,)),
            pltpu.SemaphoreType.DMA((nbuf,)),
            pltpu.SemaphoreType.DMA((nbuf,)),
            pltpu.SemaphoreType.DMA((nbuf,)),
        ],
    )
    def k(cos_hbm, sin_hbm, idx_hbm, cos_out, sin_out, idx_v, cbuf, sbuf,
          gcs, gss, wcs, wss):
        wid = lax.axis_index("s") * info.num_cores + lax.axis_index("c")
        base = wid * n_per_w
        pltpu.sync_copy(idx_hbm.at[wid], idx_v)

        cg = [None] * nbuf
        sg = [None] * nbuf
        cw = [None] * nbuf
        sw = [None] * nbuf

        def refill(r):
            s = r % nbuf
            cg[s] = pltpu.async_copy(cos_hbm.at[idx_v.at[r]], cbuf.at[s], gcs.at[s])
            sg[s] = pltpu.async_copy(sin_hbm.at[idx_v.at[r]], sbuf.at[s], gss.at[s])

        for r in range(min(look + 1, n_ch)):
            refill(r)
        for j in range(n_ch):
            s = j % nbuf
            r = j + look + 1
            if r < n_ch:
                sr = r % nbuf
                if cw[sr] is not None:
                    cw[sr].wait()
                    sw[sr].wait()
                refill(r)
            cg[s].wait()
            sg[s].wait()
            off = base + j * CHUNK
            cw[s] = pltpu.async_copy(cbuf.at[s], cos_out.at[pl.ds(off, CHUNK)], wcs.at[s])
            sw[s] = pltpu.async_copy(sbuf.at[s], sin_out.at[pl.ds(off, CHUNK)], wss.at[s])
        for s in range(nbuf):
            if cw[s] is not None:
                cw[s].wait()
                sw[s].wait()

    return k(cos_cached, sin_cached, idx)


def kernel(x, position_ids, cos_cached, sin_cached):
    b, s = position_ids.shape
    cos_flat, sin_flat = _rope_gather(position_ids, cos_cached, sin_cached)
    cos = cos_flat.reshape(b, s, DIM).astype(x.dtype)
    sin = sin_flat.reshape(b, s, DIM).astype(x.dtype)
    return cos, sin
